# Initial kernel scaffold; baseline (speedup 1.0000x reference)
import functools

import jax
import jax.numpy as jnp
from jax import lax
from jax.experimental import pallas as pl
from jax.experimental.pallas import tpu as pltpu
from jax.experimental.pallas import tpu_sc as plsc

_N = 10000
_E = 160000
_F = 512
_CW = 128
_NCHUNK = _F // _CW
_B = 128
_NB = _E // _B
_NSUB = 16
_NCORE = 2
_SLAB = _N // _NSUB

_mesh = plsc.VectorSubcoreMesh(core_axis_name="c", subcore_axis_name="s")


def _deg_body(dst_hbm, out_hbm, dst_v, ones_v, zbuf_v, acc_sh):
    cid = lax.axis_index("c")
    sid = lax.axis_index("s")

    @pl.loop(0, _B)
    def _mk1(r):
        ones_v[r, :] = jnp.ones((16,), jnp.float32)

    @pl.loop(0, 125)
    def _mk0(r):
        zbuf_v[r, :] = jnp.zeros((16,), jnp.float32)

    for core_k in range(_NCORE):
        @pl.when(cid == core_k)
        def _core():
            @pl.loop(0, 5)
            def _z(t):
                pltpu.sync_copy(zbuf_v, acc_sh.at[pl.ds(sid * _SLAB + t * 125, 125)])
            plsc.subcore_barrier()

            @pl.loop(0, 40)
            def _acc(t):
                b_local = sid + t * _NSUB
                @pl.when(b_local < _NB // 2)
                def _do():
                    eoff = (core_k * (_NB // 2) + b_local) * _B
                    pltpu.sync_copy(dst_hbm.at[pl.ds(eoff, _B)], dst_v)
                    pltpu.sync_copy(ones_v, acc_sh.at[dst_v], add=True)
            plsc.subcore_barrier()

            pltpu.sync_copy(acc_sh.at[pl.ds(sid * _SLAB, _SLAB)],
                            out_hbm.at[core_k, pl.ds(sid * _SLAB, _SLAB)])


@jax.jit
def _sc_degree(dst):
    return pl.kernel(
        _deg_body,
        out_type=jax.ShapeDtypeStruct((_NCORE, _N, 16), jnp.float32),
        mesh=_mesh,
        scratch_types=[
            pltpu.VMEM((_B,), jnp.int32),
            pltpu.VMEM((_B, 16), jnp.float32),
            pltpu.VMEM((125, 16), jnp.float32),
            pltpu.VMEM_SHARED((_N, 16), jnp.float32),
        ],
    )(dst)


def _agg_body(p_hbm, src_hbm, dst_hbm, out_hbm,
              src_v, adj_v, dst_v, rows_v, zbuf_v, acc_sh):
    cid = lax.axis_index("c")
    sid = lax.axis_index("s")

    @pl.loop(0, 125)
    def _mk0(r):
        @pl.loop(0, _CW // 16)
        def _mk0i(i):
            zbuf_v[r, pl.ds(i * 16, 16)] = jnp.zeros((16,), jnp.float32)

    for core_k in range(_NCORE):
        @pl.when(cid == core_k)
        def _core():
            for j in range(_NCHUNK // _NCORE):
                c = core_k * (_NCHUNK // _NCORE) + j
                roff = c * _N

                @pl.loop(0, 5)
                def _z(t):
                    pltpu.sync_copy(zbuf_v, acc_sh.at[pl.ds(sid * _SLAB + t * 125, 125)])
                plsc.subcore_barrier()

                @pl.loop(0, 79)
                def _acc(t):
                    b = sid + t * _NSUB
                    @pl.when(b < _NB)
                    def _do():
                        eoff = b * _B
                        pltpu.sync_copy(src_hbm.at[pl.ds(eoff, _B)], src_v)
                        pltpu.sync_copy(dst_hbm.at[pl.ds(eoff, _B)], dst_v)

                        @pl.loop(0, _B // 16)
                        def _adj(i):
                            adj_v[pl.ds(i * 16, 16)] = src_v[pl.ds(i * 16, 16)] + roff

                        pltpu.sync_copy(p_hbm.at[adj_v], rows_v)
                        pltpu.sync_copy(rows_v, acc_sh.at[dst_v], add=True)
                plsc.subcore_barrier()

                pltpu.sync_copy(acc_sh.at[pl.ds(sid * _SLAB, _SLAB)],
                                out_hbm.at[c, pl.ds(sid * _SLAB, _SLAB)])


@jax.jit
def _sc_aggregate(p_flat, src, dst):
    return pl.kernel(
        _agg_body,
        out_type=jax.ShapeDtypeStruct((_NCHUNK, _N, _CW), jnp.float32),
        mesh=_mesh,
        scratch_types=[
            pltpu.VMEM((_B,), jnp.int32),
            pltpu.VMEM((_B,), jnp.int32),
            pltpu.VMEM((_B,), jnp.int32),
            pltpu.VMEM((_B, _CW), jnp.float32),
            pltpu.VMEM((125, _CW), jnp.float32),
            pltpu.VMEM_SHARED((_N, _CW), jnp.float32),
        ],
    )(p_flat, src, dst)


_MB = 2000


def _mm_body(h_ref, wl_ref, wr_ref, b_ref, p_ref, q_ref):
    h = h_ref[...]
    for c in range(_NCHUNK):
        p_ref[c] = lax.dot_general(h, wl_ref[c], (((1,), (0,)), ((), ())),
                                   precision=lax.Precision.HIGHEST)
    q_ref[...] = lax.dot_general(h, wr_ref[...], (((1,), (0,)), ((), ())),
                                 precision=lax.Precision.HIGHEST) + b_ref[...]


def _tc_mm(h, wl_chunks, wr, b_row):
    k = h.shape[1]
    return pl.pallas_call(
        _mm_body,
        grid=(_N // _MB,),
        in_specs=[
            pl.BlockSpec((_MB, k), lambda i: (i, 0)),
            pl.BlockSpec((_NCHUNK, k, _CW), lambda i: (0, 0, 0)),
            pl.BlockSpec((k, _F), lambda i: (0, 0)),
            pl.BlockSpec((1, _F), lambda i: (0, 0)),
        ],
        out_specs=[
            pl.BlockSpec((_NCHUNK, _MB, _CW), lambda i: (0, i, 0)),
            pl.BlockSpec((_MB, _F), lambda i: (i, 0)),
        ],
        out_shape=[
            jax.ShapeDtypeStruct((_NCHUNK, _N, _CW), jnp.float32),
            jax.ShapeDtypeStruct((_N, _F), jnp.float32),
        ],
    )(h, wl_chunks, wr, b_row)


def _comb_body(agg_ref, deg_ref, q_ref, o_ref):
    deg = deg_ref[0, :, 0] + deg_ref[1, :, 0]
    rdeg = 1.0 / jnp.maximum(deg, 1.0)
    for c in range(_NCHUNK):
        v = agg_ref[c] * rdeg[:, None] + q_ref[:, c * _CW:(c + 1) * _CW]
        o_ref[:, c * _CW:(c + 1) * _CW] = jnp.maximum(v, 0.0)


def _tc_combine(agg, deg2, q):
    return pl.pallas_call(
        _comb_body,
        grid=(_N // _MB,),
        in_specs=[
            pl.BlockSpec((_NCHUNK, _MB, _CW), lambda i: (0, i, 0)),
            pl.BlockSpec((_NCORE, _MB, 16), lambda i: (0, i, 0)),
            pl.BlockSpec((_MB, _F), lambda i: (i, 0)),
        ],
        out_specs=pl.BlockSpec((_MB, _F), lambda i: (i, 0)),
        out_shape=jax.ShapeDtypeStruct((_N, _F), jnp.float32),
    )(agg, deg2, q)


def _fin_body(h_ref, w_ref, b_ref, o_ref):
    o_ref[...] = lax.dot_general(h_ref[...], w_ref[...], (((1,), (0,)), ((), ())),
                                 precision=lax.Precision.HIGHEST) + b_ref[...]


def _tc_final(h, w, b_row):
    nclass = w.shape[1]
    return pl.pallas_call(
        _fin_body,
        grid=(_N // _MB,),
        in_specs=[
            pl.BlockSpec((_MB, _F), lambda i: (i, 0)),
            pl.BlockSpec((_F, nclass), lambda i: (0, 0)),
            pl.BlockSpec((1, nclass), lambda i: (0, 0)),
        ],
        out_specs=pl.BlockSpec((_MB, nclass), lambda i: (i, 0)),
        out_shape=jax.ShapeDtypeStruct((_N, nclass), jnp.float32),
    )(h, w, b_row)


def kernel(x, edge_index, edge_attr,
           W_l1, W_r1, b1, W_l2, W_r2, b2, W_l3, W_r3, b3,
           W_l4, W_r4, b4, W_l5, W_r5, b5, W_lin, b_lin):
    src = edge_index[0].astype(jnp.int32)
    dst = edge_index[1].astype(jnp.int32)

    deg2 = _sc_degree(dst)

    h = x
    for Wl, Wr, b in ((W_l1, W_r1, b1), (W_l2, W_r2, b2), (W_l3, W_r3, b3),
                      (W_l4, W_r4, b4), (W_l5, W_r5, b5)):
        k = Wl.shape[0]
        wl_chunks = jnp.transpose(Wl.reshape(k, _NCHUNK, _CW), (1, 0, 2))
        p4, q = _tc_mm(h, wl_chunks, Wr, b.reshape(1, _F))
        agg = _sc_aggregate(p4.reshape(_NCHUNK * _N, _CW), src, dst)
        h = _tc_combine(agg, deg2, q)

    return _tc_final(h, W_lin, b_lin.reshape(1, W_lin.shape[1]))


# speedup vs baseline: 2.5734x; 2.5734x over previous
import functools

import jax
import jax.numpy as jnp
from jax import lax
from jax.experimental import pallas as pl
from jax.experimental.pallas import tpu as pltpu
from jax.experimental.pallas import tpu_sc as plsc

_N = 10000
_E = 160000
_F = 512
_CW = 128
_NCHUNK = _F // _CW
_B = 128
_NB = _E // _B
_NSUB = 16
_NCORE = 2
_NPAD = 10240
_SLAB = _NPAD // _NSUB

_mesh = plsc.VectorSubcoreMesh(core_axis_name="c", subcore_axis_name="s")


def _deg_body(dst_hbm, out_hbm, dst_v, ones_v, zbuf_v, acc_sh):
    cid = lax.axis_index("c")
    sid = lax.axis_index("s")

    @pl.loop(0, _B)
    def _mk1(r):
        @pl.loop(0, _CW // 16)
        def _mk1i(i):
            ones_v[r, pl.ds(i * 16, 16)] = jnp.ones((16,), jnp.float32)

    @pl.loop(0, _B)
    def _mk0(r):
        @pl.loop(0, _CW // 16)
        def _mk0i(i):
            zbuf_v[r, pl.ds(i * 16, 16)] = jnp.zeros((16,), jnp.float32)

    for core_k in range(_NCORE):
        @pl.when(cid == core_k)
        def _core():
            @pl.loop(0, _SLAB // _B)
            def _z(t):
                pltpu.sync_copy(zbuf_v, acc_sh.at[pl.ds(sid * _SLAB + t * _B, _B)])
            plsc.subcore_barrier()

            @pl.loop(0, 40)
            def _acc(t):
                b_local = sid + t * _NSUB
                @pl.when(b_local < _NB // 2)
                def _do():
                    eoff = (core_k * (_NB // 2) + b_local) * _B
                    pltpu.sync_copy(dst_hbm.at[pl.ds(eoff, _B)], dst_v)
                    pltpu.sync_copy(ones_v, acc_sh.at[dst_v], add=True)
            plsc.subcore_barrier()

            pltpu.sync_copy(acc_sh.at[pl.ds(sid * _SLAB, _SLAB)],
                            out_hbm.at[core_k, pl.ds(sid * _SLAB, _SLAB)])


@jax.jit
def _sc_degree(dst):
    return pl.kernel(
        _deg_body,
        out_type=jax.ShapeDtypeStruct((_NCORE, _NPAD, _CW), jnp.float32),
        mesh=_mesh,
        scratch_types=[
            pltpu.VMEM((_B,), jnp.int32),
            pltpu.VMEM((_B, _CW), jnp.float32),
            pltpu.VMEM((_B, _CW), jnp.float32),
            pltpu.VMEM_SHARED((_NPAD, _CW), jnp.float32),
        ],
    )(dst)


def _agg_body(p_hbm, src_hbm, dst_hbm, out_hbm,
              src_v, adj_v, dst_v, rows_v, zbuf_v, acc_sh):
    cid = lax.axis_index("c")
    sid = lax.axis_index("s")

    @pl.loop(0, _B)
    def _mk0(r):
        @pl.loop(0, _CW // 16)
        def _mk0i(i):
            zbuf_v[r, pl.ds(i * 16, 16)] = jnp.zeros((16,), jnp.float32)

    for core_k in range(_NCORE):
        @pl.when(cid == core_k)
        def _core():
            for j in range(_NCHUNK // _NCORE):
                c = core_k * (_NCHUNK // _NCORE) + j
                roff = c * _N

                @pl.loop(0, _SLAB // _B)
                def _z(t):
                    pltpu.sync_copy(zbuf_v, acc_sh.at[pl.ds(sid * _SLAB + t * _B, _B)])
                plsc.subcore_barrier()

                @pl.loop(0, 79)
                def _acc(t):
                    b = sid + t * _NSUB
                    @pl.when(b < _NB)
                    def _do():
                        eoff = b * _B
                        pltpu.sync_copy(src_hbm.at[pl.ds(eoff, _B)], src_v)
                        pltpu.sync_copy(dst_hbm.at[pl.ds(eoff, _B)], dst_v)

                        @pl.loop(0, _B // 16)
                        def _adj(i):
                            adj_v[pl.ds(i * 16, 16)] = src_v[pl.ds(i * 16, 16)] + roff

                        pltpu.sync_copy(p_hbm.at[adj_v], rows_v)
                        pltpu.sync_copy(rows_v, acc_sh.at[dst_v], add=True)
                plsc.subcore_barrier()

                pltpu.sync_copy(acc_sh.at[pl.ds(sid * _SLAB, _SLAB)],
                                out_hbm.at[c, pl.ds(sid * _SLAB, _SLAB)])


@jax.jit
def _sc_aggregate(p_flat, src, dst):
    return pl.kernel(
        _agg_body,
        out_type=jax.ShapeDtypeStruct((_NCHUNK, _NPAD, _CW), jnp.float32),
        mesh=_mesh,
        scratch_types=[
            pltpu.VMEM((_B,), jnp.int32),
            pltpu.VMEM((_B,), jnp.int32),
            pltpu.VMEM((_B,), jnp.int32),
            pltpu.VMEM((_B, _CW), jnp.float32),
            pltpu.VMEM((_B, _CW), jnp.float32),
            pltpu.VMEM_SHARED((_NPAD, _CW), jnp.float32),
        ],
    )(p_flat, src, dst)


_MB = 2000


def _mm_body(h_ref, wl_ref, wr_ref, b_ref, p_ref, q_ref):
    h = h_ref[...]
    for c in range(_NCHUNK):
        p_ref[c] = lax.dot_general(h, wl_ref[c], (((1,), (0,)), ((), ())),
                                   precision=lax.Precision.HIGHEST)
    q_ref[...] = lax.dot_general(h, wr_ref[...], (((1,), (0,)), ((), ())),
                                 precision=lax.Precision.HIGHEST) + b_ref[...]


def _tc_mm(h, wl_chunks, wr, b_row):
    k = h.shape[1]
    return pl.pallas_call(
        _mm_body,
        grid=(_N // _MB,),
        in_specs=[
            pl.BlockSpec((_MB, k), lambda i: (i, 0)),
            pl.BlockSpec((_NCHUNK, k, _CW), lambda i: (0, 0, 0)),
            pl.BlockSpec((k, _F), lambda i: (0, 0)),
            pl.BlockSpec((1, _F), lambda i: (0, 0)),
        ],
        out_specs=[
            pl.BlockSpec((_NCHUNK, _MB, _CW), lambda i: (0, i, 0)),
            pl.BlockSpec((_MB, _F), lambda i: (i, 0)),
        ],
        out_shape=[
            jax.ShapeDtypeStruct((_NCHUNK, _N, _CW), jnp.float32),
            jax.ShapeDtypeStruct((_N, _F), jnp.float32),
        ],
    )(h, wl_chunks, wr, b_row)


def _comb_body(agg_ref, deg_ref, q_ref, o_ref):
    deg = deg_ref[0, :, 0] + deg_ref[1, :, 0]
    rdeg = 1.0 / jnp.maximum(deg, 1.0)
    for c in range(_NCHUNK):
        v = agg_ref[c] * rdeg[:, None] + q_ref[:, c * _CW:(c + 1) * _CW]
        o_ref[:, c * _CW:(c + 1) * _CW] = jnp.maximum(v, 0.0)


def _tc_combine(agg, deg2, q):
    return pl.pallas_call(
        _comb_body,
        grid=(_N // _MB,),
        in_specs=[
            pl.BlockSpec((_NCHUNK, _MB, _CW), lambda i: (0, i, 0)),
            pl.BlockSpec((_NCORE, _MB, _CW), lambda i: (0, i, 0)),
            pl.BlockSpec((_MB, _F), lambda i: (i, 0)),
        ],
        out_specs=pl.BlockSpec((_MB, _F), lambda i: (i, 0)),
        out_shape=jax.ShapeDtypeStruct((_N, _F), jnp.float32),
    )(agg, deg2, q)


def _fin_body(h_ref, w_ref, b_ref, o_ref):
    o_ref[...] = lax.dot_general(h_ref[...], w_ref[...], (((1,), (0,)), ((), ())),
                                 precision=lax.Precision.HIGHEST) + b_ref[...]


def _tc_final(h, w, b_row):
    nclass = w.shape[1]
    return pl.pallas_call(
        _fin_body,
        grid=(_N // _MB,),
        in_specs=[
            pl.BlockSpec((_MB, _F), lambda i: (i, 0)),
            pl.BlockSpec((_F, nclass), lambda i: (0, 0)),
            pl.BlockSpec((1, nclass), lambda i: (0, 0)),
        ],
        out_specs=pl.BlockSpec((_MB, nclass), lambda i: (i, 0)),
        out_shape=jax.ShapeDtypeStruct((_N, nclass), jnp.float32),
    )(h, w, b_row)


def kernel(x, edge_index, edge_attr,
           W_l1, W_r1, b1, W_l2, W_r2, b2, W_l3, W_r3, b3,
           W_l4, W_r4, b4, W_l5, W_r5, b5, W_lin, b_lin):
    src = edge_index[0].astype(jnp.int32)
    dst = edge_index[1].astype(jnp.int32)

    deg2 = _sc_degree(dst)

    h = x
    for Wl, Wr, b in ((W_l1, W_r1, b1), (W_l2, W_r2, b2), (W_l3, W_r3, b3),
                      (W_l4, W_r4, b4), (W_l5, W_r5, b5)):
        k = Wl.shape[0]
        wl_chunks = jnp.transpose(Wl.reshape(k, _NCHUNK, _CW), (1, 0, 2))
        p4, q = _tc_mm(h, wl_chunks, Wr, b.reshape(1, _F))
        agg = _sc_aggregate(p4.reshape(_NCHUNK * _N, _CW), src, dst)
        h = _tc_combine(agg, deg2, q)

    return _tc_final(h, W_lin, b_lin.reshape(1, W_lin.shape[1]))


# speedup vs baseline: 3.8006x; 1.4769x over previous
import functools

import jax
import jax.numpy as jnp
from jax import lax
from jax.experimental import pallas as pl
from jax.experimental.pallas import tpu as pltpu
from jax.experimental.pallas import tpu_sc as plsc

_N = 10000
_E = 160000
_F = 512
_CW = 128
_NCHUNK = _F // _CW
_B = 128
_NB = _E // _B
_NSUB = 16
_NCORE = 2
_NPAD = 10240
_SLAB = _NPAD // _NSUB

_mesh = plsc.VectorSubcoreMesh(core_axis_name="c", subcore_axis_name="s")


def _deg_body(dst_hbm, out_hbm, dst_v, ones_v, zbuf_v, acc_sh):
    cid = lax.axis_index("c")
    sid = lax.axis_index("s")

    @pl.loop(0, _B)
    def _mk1(r):
        @pl.loop(0, _CW // 16)
        def _mk1i(i):
            ones_v[r, pl.ds(i * 16, 16)] = jnp.ones((16,), jnp.float32)

    @pl.loop(0, _B)
    def _mk0(r):
        @pl.loop(0, _CW // 16)
        def _mk0i(i):
            zbuf_v[r, pl.ds(i * 16, 16)] = jnp.zeros((16,), jnp.float32)

    for core_k in range(_NCORE):
        @pl.when(cid == core_k)
        def _core():
            @pl.loop(0, _SLAB // _B)
            def _z(t):
                pltpu.sync_copy(zbuf_v, acc_sh.at[pl.ds(sid * _SLAB + t * _B, _B)])
            plsc.subcore_barrier()

            @pl.loop(0, 40)
            def _acc(t):
                b_local = sid + t * _NSUB
                @pl.when(b_local < _NB // 2)
                def _do():
                    eoff = (core_k * (_NB // 2) + b_local) * _B
                    pltpu.sync_copy(dst_hbm.at[pl.ds(eoff, _B)], dst_v)
                    pltpu.sync_copy(ones_v, acc_sh.at[dst_v], add=True)
            plsc.subcore_barrier()

            pltpu.sync_copy(acc_sh.at[pl.ds(sid * _SLAB, _SLAB)],
                            out_hbm.at[core_k, pl.ds(sid * _SLAB, _SLAB)])


@jax.jit
def _sc_degree(dst):
    return pl.kernel(
        _deg_body,
        out_type=jax.ShapeDtypeStruct((_NCORE, _NPAD, _CW), jnp.float32),
        mesh=_mesh,
        scratch_types=[
            pltpu.VMEM((_B,), jnp.int32),
            pltpu.VMEM((_B, _CW), jnp.float32),
            pltpu.VMEM((_B, _CW), jnp.float32),
            pltpu.VMEM_SHARED((_NPAD, _CW), jnp.float32),
        ],
    )(dst)


def _agg_body(p_hbm, src_hbm, dst_hbm, out_hbm,
              srcb0_v, dstb0_v, srcb1_v, dstb1_v,
              adj0_v, adj1_v, dstv0_v, dstv1_v,
              rows0_v, rows1_v, acc_sh,
              isem0, isem1, gsem0, gsem1, ssem0, ssem1):
    cid = lax.axis_index("c")
    sid = lax.axis_index("s")
    nb = jnp.where(sid < 2, 79, 78)
    bstart = sid * 78 + jnp.minimum(sid, 2)

    sets = ((srcb0_v, dstb0_v, adj0_v, dstv0_v, rows0_v, isem0, gsem0, ssem0),
            (srcb1_v, dstb1_v, adj1_v, dstv1_v, rows1_v, isem1, gsem1, ssem1))

    def _issue_idx(b, s):
        srcb, dstb = s[0], s[1]
        pltpu.async_copy(src_hbm.at[bstart + b], srcb, s[5])
        pltpu.async_copy(dst_hbm.at[bstart + b], dstb, s[5])

    def _wait_idx(s):
        pltpu.make_async_copy(src_hbm.at[0], s[0], s[5]).wait()
        pltpu.make_async_copy(dst_hbm.at[0], s[1], s[5]).wait()

    def _wait_scat(s):
        pltpu.make_async_copy(s[4], acc_sh.at[s[3]], s[7]).wait()

    for core_k in range(_NCORE):
        @pl.when(cid == core_k)
        def _core():
            for j in range(_NCHUNK // _NCORE):
                c = core_k * (_NCHUNK // _NCORE) + j
                roff = c * _N

                _issue_idx(0, sets[0])
                _issue_idx(1, sets[1])

                @pl.loop(0, _B)
                def _mk0(r):
                    @pl.loop(0, _CW // 16)
                    def _mk0i(i):
                        rows0_v[r, pl.ds(i * 16, 16)] = jnp.zeros((16,), jnp.float32)

                @pl.loop(0, _SLAB // _B)
                def _z(t):
                    pltpu.sync_copy(rows0_v, acc_sh.at[pl.ds(sid * _SLAB + t * _B, _B)])
                plsc.subcore_barrier()

                def _gather(b, s, ws_pred):
                    _wait_idx(s)
                    @pl.loop(0, _B // 16)
                    def _adj(i):
                        s[2][pl.ds(i * 16, 16)] = s[0][0, pl.ds(i * 16, 16)] + roff
                    if ws_pred is None:
                        _wait_scat(s)
                    else:
                        @pl.when(ws_pred)
                        def _ws():
                            _wait_scat(s)
                    pltpu.async_copy(p_hbm.at[s[2]], s[4], s[6])

                def _scatter(b, s):
                    pltpu.make_async_copy(p_hbm.at[s[2]], s[4], s[6]).wait()
                    @pl.loop(0, _B // 16)
                    def _d(i):
                        s[3][pl.ds(i * 16, 16)] = s[1][0, pl.ds(i * 16, 16)]
                    pltpu.async_copy(s[4], acc_sh.at[s[3]], s[7], add=True)
                    @pl.when(b + 2 < nb)
                    def _ni():
                        _issue_idx(b + 2, s)

                @pl.loop(0, 39)
                def _acc(t):
                    b0 = 2 * t
                    _gather(b0, sets[0], t > 0)
                    _gather(b0 + 1, sets[1], t > 0)
                    _scatter(b0, sets[0])
                    _scatter(b0 + 1, sets[1])

                @pl.when(nb > 78)
                def _tail():
                    _gather(78, sets[0], None)
                    _scatter(78, sets[0])

                _wait_scat(sets[0])
                _wait_scat(sets[1])
                plsc.subcore_barrier()

                pltpu.sync_copy(acc_sh.at[pl.ds(sid * _SLAB, _SLAB)],
                                out_hbm.at[c, pl.ds(sid * _SLAB, _SLAB)])


@jax.jit
def _sc_aggregate(p_flat, src, dst):
    src3 = src.reshape(_NB, 1, _B)
    dst3 = dst.reshape(_NB, 1, _B)
    return pl.kernel(
        _agg_body,
        out_type=jax.ShapeDtypeStruct((_NCHUNK, _NPAD, _CW), jnp.float32),
        mesh=_mesh,
        scratch_types=[
            pltpu.VMEM((1, _B), jnp.int32),
            pltpu.VMEM((1, _B), jnp.int32),
            pltpu.VMEM((1, _B), jnp.int32),
            pltpu.VMEM((1, _B), jnp.int32),
            pltpu.VMEM((_B,), jnp.int32),
            pltpu.VMEM((_B,), jnp.int32),
            pltpu.VMEM((_B,), jnp.int32),
            pltpu.VMEM((_B,), jnp.int32),
            pltpu.VMEM((_B, _CW), jnp.float32),
            pltpu.VMEM((_B, _CW), jnp.float32),
            pltpu.VMEM_SHARED((_NPAD, _CW), jnp.float32),
            pltpu.SemaphoreType.DMA,
            pltpu.SemaphoreType.DMA,
            pltpu.SemaphoreType.DMA,
            pltpu.SemaphoreType.DMA,
            pltpu.SemaphoreType.DMA,
            pltpu.SemaphoreType.DMA,
        ],
    )(p_flat, src3, dst3)


_MB = 2000


def _mm_body(h_ref, wl_ref, wr_ref, b_ref, p_ref, q_ref):
    h = h_ref[...]
    for c in range(_NCHUNK):
        p_ref[c] = lax.dot_general(h, wl_ref[c], (((1,), (0,)), ((), ())),
                                   precision=lax.Precision.HIGHEST)
    q_ref[...] = lax.dot_general(h, wr_ref[...], (((1,), (0,)), ((), ())),
                                 precision=lax.Precision.HIGHEST) + b_ref[...]


def _tc_mm(h, wl_chunks, wr, b_row):
    k = h.shape[1]
    return pl.pallas_call(
        _mm_body,
        grid=(_N // _MB,),
        in_specs=[
            pl.BlockSpec((_MB, k), lambda i: (i, 0)),
            pl.BlockSpec((_NCHUNK, k, _CW), lambda i: (0, 0, 0)),
            pl.BlockSpec((k, _F), lambda i: (0, 0)),
            pl.BlockSpec((1, _F), lambda i: (0, 0)),
        ],
        out_specs=[
            pl.BlockSpec((_NCHUNK, _MB, _CW), lambda i: (0, i, 0)),
            pl.BlockSpec((_MB, _F), lambda i: (i, 0)),
        ],
        out_shape=[
            jax.ShapeDtypeStruct((_NCHUNK, _N, _CW), jnp.float32),
            jax.ShapeDtypeStruct((_N, _F), jnp.float32),
        ],
    )(h, wl_chunks, wr, b_row)


def _comb_body(agg_ref, deg_ref, q_ref, o_ref):
    deg = deg_ref[0, :, 0] + deg_ref[1, :, 0]
    rdeg = 1.0 / jnp.maximum(deg, 1.0)
    for c in range(_NCHUNK):
        v = agg_ref[c] * rdeg[:, None] + q_ref[:, c * _CW:(c + 1) * _CW]
        o_ref[:, c * _CW:(c + 1) * _CW] = jnp.maximum(v, 0.0)


def _tc_combine(agg, deg2, q):
    return pl.pallas_call(
        _comb_body,
        grid=(_N // _MB,),
        in_specs=[
            pl.BlockSpec((_NCHUNK, _MB, _CW), lambda i: (0, i, 0)),
            pl.BlockSpec((_NCORE, _MB, _CW), lambda i: (0, i, 0)),
            pl.BlockSpec((_MB, _F), lambda i: (i, 0)),
        ],
        out_specs=pl.BlockSpec((_MB, _F), lambda i: (i, 0)),
        out_shape=jax.ShapeDtypeStruct((_N, _F), jnp.float32),
    )(agg, deg2, q)


def _fin_body(h_ref, w_ref, b_ref, o_ref):
    o_ref[...] = lax.dot_general(h_ref[...], w_ref[...], (((1,), (0,)), ((), ())),
                                 precision=lax.Precision.HIGHEST) + b_ref[...]


def _tc_final(h, w, b_row):
    nclass = w.shape[1]
    return pl.pallas_call(
        _fin_body,
        grid=(_N // _MB,),
        in_specs=[
            pl.BlockSpec((_MB, _F), lambda i: (i, 0)),
            pl.BlockSpec((_F, nclass), lambda i: (0, 0)),
            pl.BlockSpec((1, nclass), lambda i: (0, 0)),
        ],
        out_specs=pl.BlockSpec((_MB, nclass), lambda i: (i, 0)),
        out_shape=jax.ShapeDtypeStruct((_N, nclass), jnp.float32),
    )(h, w, b_row)


def kernel(x, edge_index, edge_attr,
           W_l1, W_r1, b1, W_l2, W_r2, b2, W_l3, W_r3, b3,
           W_l4, W_r4, b4, W_l5, W_r5, b5, W_lin, b_lin):
    src = edge_index[0].astype(jnp.int32)
    dst = edge_index[1].astype(jnp.int32)

    deg2 = _sc_degree(dst)

    h = x
    for Wl, Wr, b in ((W_l1, W_r1, b1), (W_l2, W_r2, b2), (W_l3, W_r3, b3),
                      (W_l4, W_r4, b4), (W_l5, W_r5, b5)):
        k = Wl.shape[0]
        wl_chunks = jnp.transpose(Wl.reshape(k, _NCHUNK, _CW), (1, 0, 2))
        p4, q = _tc_mm(h, wl_chunks, Wr, b.reshape(1, _F))
        agg = _sc_aggregate(p4.reshape(_NCHUNK * _N, _CW), src, dst)
        h = _tc_combine(agg, deg2, q)

    return _tc_final(h, W_lin, b_lin.reshape(1, W_lin.shape[1]))


# speedup vs baseline: 4.6857x; 1.2329x over previous
import functools

import jax
import jax.numpy as jnp
from jax import lax
from jax.experimental import pallas as pl
from jax.experimental.pallas import tpu as pltpu
from jax.experimental.pallas import tpu_sc as plsc

_N = 10000
_E = 160000
_F = 512
_CW = 128
_NCHUNK = _F // _CW
_B = 128
_NB = _E // _B
_NSUB = 16
_NCORE = 2
_NPAD = 10240
_SLAB = _NPAD // _NSUB

_mesh = plsc.VectorSubcoreMesh(core_axis_name="c", subcore_axis_name="s")


def _deg_body(dst_hbm, out_hbm, dst_v, ones_v, zbuf_v, acc_sh):
    cid = lax.axis_index("c")
    sid = lax.axis_index("s")

    @pl.loop(0, _B)
    def _mk1(r):
        @pl.loop(0, _CW // 16)
        def _mk1i(i):
            ones_v[r, pl.ds(i * 16, 16)] = jnp.ones((16,), jnp.float32)

    @pl.loop(0, _B)
    def _mk0(r):
        @pl.loop(0, _CW // 16)
        def _mk0i(i):
            zbuf_v[r, pl.ds(i * 16, 16)] = jnp.zeros((16,), jnp.float32)

    for core_k in range(_NCORE):
        @pl.when(cid == core_k)
        def _core():
            @pl.loop(0, _SLAB // _B)
            def _z(t):
                pltpu.sync_copy(zbuf_v, acc_sh.at[pl.ds(sid * _SLAB + t * _B, _B)])
            plsc.subcore_barrier()

            @pl.loop(0, 40)
            def _acc(t):
                b_local = sid + t * _NSUB
                @pl.when(b_local < _NB // 2)
                def _do():
                    eoff = (core_k * (_NB // 2) + b_local) * _B
                    pltpu.sync_copy(dst_hbm.at[pl.ds(eoff, _B)], dst_v)
                    pltpu.sync_copy(ones_v, acc_sh.at[dst_v], add=True)
            plsc.subcore_barrier()

            pltpu.sync_copy(acc_sh.at[pl.ds(sid * _SLAB, _SLAB)],
                            out_hbm.at[core_k, pl.ds(sid * _SLAB, _SLAB)])


@jax.jit
def _sc_degree(dst):
    return pl.kernel(
        _deg_body,
        out_type=jax.ShapeDtypeStruct((_NCORE, _NPAD, _CW), jnp.float32),
        mesh=_mesh,
        scratch_types=[
            pltpu.VMEM((_B,), jnp.int32),
            pltpu.VMEM((_B, _CW), jnp.float32),
            pltpu.VMEM((_B, _CW), jnp.float32),
            pltpu.VMEM_SHARED((_NPAD, _CW), jnp.float32),
        ],
    )(dst)


def _agg_body(p_hbm, src_hbm, dst_hbm, out_hbm,
              srcb0_v, dstb0_v, srcb1_v, dstb1_v,
              adj0_v, adj1_v, dstv0_v, dstv1_v,
              rows0_v, rows1_v, acc_sh,
              isem0, isem1, gsem0, gsem1, ssem0, ssem1):
    cid = lax.axis_index("c")
    sid = lax.axis_index("s")
    nb = jnp.where(sid < 2, 79, 78)
    bstart = sid * 78 + jnp.minimum(sid, 2)

    sets = ((srcb0_v, dstb0_v, adj0_v, dstv0_v, rows0_v, isem0, gsem0, ssem0),
            (srcb1_v, dstb1_v, adj1_v, dstv1_v, rows1_v, isem1, gsem1, ssem1))

    def _issue_idx(b, s):
        srcb, dstb = s[0], s[1]
        pltpu.async_copy(src_hbm.at[bstart + b], srcb, s[5])
        pltpu.async_copy(dst_hbm.at[bstart + b], dstb, s[5])

    def _wait_idx(s):
        pltpu.make_async_copy(src_hbm.at[0], s[0], s[5]).wait()
        pltpu.make_async_copy(dst_hbm.at[0], s[1], s[5]).wait()

    def _wait_scat(s):
        pltpu.make_async_copy(s[4], acc_sh.at[s[3]], s[7]).wait()

    for core_k in range(_NCORE):
        @pl.when(cid == core_k)
        def _core():
            for j in range(_NCHUNK // _NCORE):
                c = core_k * (_NCHUNK // _NCORE) + j
                roff = c * _N

                _issue_idx(0, sets[0])
                _issue_idx(1, sets[1])

                @pl.loop(0, _B)
                def _mk0(r):
                    @pl.loop(0, _CW // 16)
                    def _mk0i(i):
                        rows0_v[r, pl.ds(i * 16, 16)] = jnp.zeros((16,), jnp.float32)

                @pl.loop(0, _SLAB // _B)
                def _z(t):
                    pltpu.sync_copy(rows0_v, acc_sh.at[pl.ds(sid * _SLAB + t * _B, _B)])
                plsc.subcore_barrier()

                def _gather(b, s, ws_pred):
                    _wait_idx(s)
                    @pl.loop(0, _B // 16)
                    def _adj(i):
                        s[2][pl.ds(i * 16, 16)] = s[0][0, pl.ds(i * 16, 16)] + roff
                    if ws_pred is None:
                        _wait_scat(s)
                    else:
                        @pl.when(ws_pred)
                        def _ws():
                            _wait_scat(s)
                    pltpu.async_copy(p_hbm.at[s[2]], s[4], s[6])

                def _scatter(b, s):
                    pltpu.make_async_copy(p_hbm.at[s[2]], s[4], s[6]).wait()
                    @pl.loop(0, _B // 16)
                    def _d(i):
                        s[3][pl.ds(i * 16, 16)] = s[1][0, pl.ds(i * 16, 16)]
                    pltpu.async_copy(s[4], acc_sh.at[s[3]], s[7], add=True)
                    @pl.when(b + 2 < nb)
                    def _ni():
                        _issue_idx(b + 2, s)

                @pl.loop(0, 39)
                def _acc(t):
                    b0 = 2 * t
                    _gather(b0, sets[0], t > 0)
                    _gather(b0 + 1, sets[1], t > 0)
                    _scatter(b0, sets[0])
                    _scatter(b0 + 1, sets[1])

                @pl.when(nb > 78)
                def _tail():
                    _gather(78, sets[0], None)
                    _scatter(78, sets[0])

                _wait_scat(sets[0])
                _wait_scat(sets[1])
                plsc.subcore_barrier()

                pltpu.sync_copy(acc_sh.at[pl.ds(sid * _SLAB, _SLAB)],
                                out_hbm.at[c, pl.ds(sid * _SLAB, _SLAB)])


@jax.jit
def _sc_aggregate(p_flat, src, dst):
    src3 = src.reshape(_NB, 1, _B)
    dst3 = dst.reshape(_NB, 1, _B)
    return pl.kernel(
        _agg_body,
        out_type=jax.ShapeDtypeStruct((_NCHUNK, _NPAD, _CW), jnp.float32),
        mesh=_mesh,
        scratch_types=[
            pltpu.VMEM((1, _B), jnp.int32),
            pltpu.VMEM((1, _B), jnp.int32),
            pltpu.VMEM((1, _B), jnp.int32),
            pltpu.VMEM((1, _B), jnp.int32),
            pltpu.VMEM((_B,), jnp.int32),
            pltpu.VMEM((_B,), jnp.int32),
            pltpu.VMEM((_B,), jnp.int32),
            pltpu.VMEM((_B,), jnp.int32),
            pltpu.VMEM((_B, _CW), jnp.float32),
            pltpu.VMEM((_B, _CW), jnp.float32),
            pltpu.VMEM_SHARED((_NPAD, _CW), jnp.float32),
            pltpu.SemaphoreType.DMA,
            pltpu.SemaphoreType.DMA,
            pltpu.SemaphoreType.DMA,
            pltpu.SemaphoreType.DMA,
            pltpu.SemaphoreType.DMA,
            pltpu.SemaphoreType.DMA,
        ],
    )(p_flat, src3, dst3)


_MB = 2000


def _mm_body(h_ref, wl_ref, wr_ref, b_ref, p_ref, q_ref):
    h = h_ref[...]
    for c in range(_NCHUNK):
        p_ref[c] = lax.dot_general(h, wl_ref[c], (((1,), (0,)), ((), ())),
                                   precision=lax.Precision.DEFAULT)
    q_ref[...] = lax.dot_general(h, wr_ref[...], (((1,), (0,)), ((), ())),
                                 precision=lax.Precision.DEFAULT) + b_ref[...]


def _tc_mm(h, wl_chunks, wr, b_row):
    k = h.shape[1]
    return pl.pallas_call(
        _mm_body,
        grid=(_N // _MB,),
        in_specs=[
            pl.BlockSpec((_MB, k), lambda i: (i, 0)),
            pl.BlockSpec((_NCHUNK, k, _CW), lambda i: (0, 0, 0)),
            pl.BlockSpec((k, _F), lambda i: (0, 0)),
            pl.BlockSpec((1, _F), lambda i: (0, 0)),
        ],
        out_specs=[
            pl.BlockSpec((_NCHUNK, _MB, _CW), lambda i: (0, i, 0)),
            pl.BlockSpec((_MB, _F), lambda i: (i, 0)),
        ],
        out_shape=[
            jax.ShapeDtypeStruct((_NCHUNK, _N, _CW), jnp.float32),
            jax.ShapeDtypeStruct((_N, _F), jnp.float32),
        ],
    )(h, wl_chunks, wr, b_row)


def _comb_body(agg_ref, deg_ref, q_ref, o_ref):
    deg = deg_ref[0, :, 0] + deg_ref[1, :, 0]
    rdeg = 1.0 / jnp.maximum(deg, 1.0)
    for c in range(_NCHUNK):
        v = agg_ref[c] * rdeg[:, None] + q_ref[:, c * _CW:(c + 1) * _CW]
        o_ref[:, c * _CW:(c + 1) * _CW] = jnp.maximum(v, 0.0)


def _tc_combine(agg, deg2, q):
    return pl.pallas_call(
        _comb_body,
        grid=(_N // _MB,),
        in_specs=[
            pl.BlockSpec((_NCHUNK, _MB, _CW), lambda i: (0, i, 0)),
            pl.BlockSpec((_NCORE, _MB, _CW), lambda i: (0, i, 0)),
            pl.BlockSpec((_MB, _F), lambda i: (i, 0)),
        ],
        out_specs=pl.BlockSpec((_MB, _F), lambda i: (i, 0)),
        out_shape=jax.ShapeDtypeStruct((_N, _F), jnp.float32),
    )(agg, deg2, q)


def _fin_body(h_ref, w_ref, b_ref, o_ref):
    o_ref[...] = lax.dot_general(h_ref[...], w_ref[...], (((1,), (0,)), ((), ())),
                                 precision=lax.Precision.DEFAULT) + b_ref[...]


def _tc_final(h, w, b_row):
    nclass = w.shape[1]
    return pl.pallas_call(
        _fin_body,
        grid=(_N // _MB,),
        in_specs=[
            pl.BlockSpec((_MB, _F), lambda i: (i, 0)),
            pl.BlockSpec((_F, nclass), lambda i: (0, 0)),
            pl.BlockSpec((1, nclass), lambda i: (0, 0)),
        ],
        out_specs=pl.BlockSpec((_MB, nclass), lambda i: (i, 0)),
        out_shape=jax.ShapeDtypeStruct((_N, nclass), jnp.float32),
    )(h, w, b_row)


def kernel(x, edge_index, edge_attr,
           W_l1, W_r1, b1, W_l2, W_r2, b2, W_l3, W_r3, b3,
           W_l4, W_r4, b4, W_l5, W_r5, b5, W_lin, b_lin):
    src = edge_index[0].astype(jnp.int32)
    dst = edge_index[1].astype(jnp.int32)

    deg2 = _sc_degree(dst)

    h = x
    for Wl, Wr, b in ((W_l1, W_r1, b1), (W_l2, W_r2, b2), (W_l3, W_r3, b3),
                      (W_l4, W_r4, b4), (W_l5, W_r5, b5)):
        k = Wl.shape[0]
        wl_chunks = jnp.transpose(Wl.reshape(k, _NCHUNK, _CW), (1, 0, 2))
        p4, q = _tc_mm(h, wl_chunks, Wr, b.reshape(1, _F))
        agg = _sc_aggregate(p4.reshape(_NCHUNK * _N, _CW), src, dst)
        h = _tc_combine(agg, deg2, q)

    return _tc_final(h, W_lin, b_lin.reshape(1, W_lin.shape[1]))


# speedup vs baseline: 4.9809x; 1.0630x over previous
import functools

import jax
import jax.numpy as jnp
from jax import lax
from jax.experimental import pallas as pl
from jax.experimental.pallas import tpu as pltpu
from jax.experimental.pallas import tpu_sc as plsc

_N = 10000
_E = 160000
_F = 512
_CW = 128
_NCHUNK = _F // _CW
_B = 128
_NB = _E // _B
_NSUB = 16
_NCORE = 2
_NPAD = 10240
_SLAB = _NPAD // _NSUB

_mesh = plsc.VectorSubcoreMesh(core_axis_name="c", subcore_axis_name="s")


def _deg_body(dst_hbm, out_hbm,
              dstb0_v, dstb1_v, dstv0_v, dstv1_v, ones_v, acc_sh,
              isem0, isem1, ssem0, ssem1):
    cid = lax.axis_index("c")
    sid = lax.axis_index("s")
    nb = jnp.where(sid < 1, 40, 39)
    bstart = sid * 39 + jnp.minimum(sid, 1)

    sets = ((dstb0_v, dstv0_v, isem0, ssem0),
            (dstb1_v, dstv1_v, isem1, ssem1))

    for core_k in range(_NCORE):
        @pl.when(cid == core_k)
        def _core():
            gb0 = core_k * (_NB // 2) + bstart

            def _issue_idx(b, s):
                pltpu.async_copy(dst_hbm.at[gb0 + b], s[0], s[2])

            def _wait_scat(s):
                pltpu.make_async_copy(ones_v, acc_sh.at[s[1]], s[3]).wait()

            _issue_idx(0, sets[0])
            _issue_idx(1, sets[1])

            @pl.loop(0, _B)
            def _mk0(r):
                @pl.loop(0, _CW // 16)
                def _mk0i(i):
                    ones_v[r, pl.ds(i * 16, 16)] = jnp.zeros((16,), jnp.float32)

            @pl.loop(0, _SLAB // _B)
            def _z(t):
                pltpu.sync_copy(ones_v, acc_sh.at[pl.ds(sid * _SLAB + t * _B, _B)])
            plsc.subcore_barrier()

            @pl.loop(0, _B)
            def _mk1(r):
                @pl.loop(0, _CW // 16)
                def _mk1i(i):
                    ones_v[r, pl.ds(i * 16, 16)] = jnp.ones((16,), jnp.float32)

            def _step(b, s, ws_pred):
                pltpu.make_async_copy(dst_hbm.at[0], s[0], s[2]).wait()
                @pl.loop(0, _B // 16)
                def _d(i):
                    s[1][pl.ds(i * 16, 16)] = s[0][0, pl.ds(i * 16, 16)]
                if ws_pred is not None:
                    @pl.when(ws_pred)
                    def _ws():
                        _wait_scat(s)
                pltpu.async_copy(ones_v, acc_sh.at[s[1]], s[3], add=True)
                @pl.when(b + 2 < nb)
                def _ni():
                    _issue_idx(b + 2, s)

            @pl.loop(0, 19)
            def _acc(t):
                b0 = 2 * t
                _step(b0, sets[0], t > 0)
                _step(b0 + 1, sets[1], t > 0)

            _step(38, sets[0], jnp.bool_(True))
            @pl.when(nb > 39)
            def _tail():
                _step(39, sets[1], jnp.bool_(True))

            _wait_scat(sets[0])
            _wait_scat(sets[1])
            plsc.subcore_barrier()

            pltpu.sync_copy(acc_sh.at[pl.ds(sid * _SLAB, _SLAB)],
                            out_hbm.at[core_k, pl.ds(sid * _SLAB, _SLAB)])


@jax.jit
def _sc_degree(dst):
    dst3 = dst.reshape(_NB, 1, _B)
    return pl.kernel(
        _deg_body,
        out_type=jax.ShapeDtypeStruct((_NCORE, _NPAD, _CW), jnp.float32),
        mesh=_mesh,
        scratch_types=[
            pltpu.VMEM((1, _B), jnp.int32),
            pltpu.VMEM((1, _B), jnp.int32),
            pltpu.VMEM((_B,), jnp.int32),
            pltpu.VMEM((_B,), jnp.int32),
            pltpu.VMEM((_B, _CW), jnp.float32),
            pltpu.VMEM_SHARED((_NPAD, _CW), jnp.float32),
            pltpu.SemaphoreType.DMA,
            pltpu.SemaphoreType.DMA,
            pltpu.SemaphoreType.DMA,
            pltpu.SemaphoreType.DMA,
        ],
    )(dst3)


def _agg_body(p_hbm, src_hbm, dst_hbm, out_hbm,
              srcb0_v, dstb0_v, srcb1_v, dstb1_v,
              adj0_v, adj1_v, dstv0_v, dstv1_v,
              rows0_v, rows1_v, acc_sh,
              isem0, isem1, gsem0, gsem1, ssem0, ssem1):
    cid = lax.axis_index("c")
    sid = lax.axis_index("s")
    nb = jnp.where(sid < 2, 79, 78)
    bstart = sid * 78 + jnp.minimum(sid, 2)

    sets = ((srcb0_v, dstb0_v, adj0_v, dstv0_v, rows0_v, isem0, gsem0, ssem0),
            (srcb1_v, dstb1_v, adj1_v, dstv1_v, rows1_v, isem1, gsem1, ssem1))

    def _issue_idx(b, s):
        srcb, dstb = s[0], s[1]
        pltpu.async_copy(src_hbm.at[bstart + b], srcb, s[5])
        pltpu.async_copy(dst_hbm.at[bstart + b], dstb, s[5])

    def _wait_idx(s):
        pltpu.make_async_copy(src_hbm.at[0], s[0], s[5]).wait()
        pltpu.make_async_copy(dst_hbm.at[0], s[1], s[5]).wait()

    def _wait_scat(s):
        pltpu.make_async_copy(s[4], acc_sh.at[s[3]], s[7]).wait()

    for core_k in range(_NCORE):
        @pl.when(cid == core_k)
        def _core():
            for j in range(_NCHUNK // _NCORE):
                c = core_k * (_NCHUNK // _NCORE) + j
                roff = c * _N

                _issue_idx(0, sets[0])
                _issue_idx(1, sets[1])

                @pl.loop(0, _B)
                def _mk0(r):
                    @pl.loop(0, _CW // 16)
                    def _mk0i(i):
                        rows0_v[r, pl.ds(i * 16, 16)] = jnp.zeros((16,), jnp.float32)

                @pl.loop(0, _SLAB // _B)
                def _z(t):
                    pltpu.sync_copy(rows0_v, acc_sh.at[pl.ds(sid * _SLAB + t * _B, _B)])
                plsc.subcore_barrier()

                def _gather(b, s, ws_pred):
                    _wait_idx(s)
                    @pl.loop(0, _B // 16)
                    def _adj(i):
                        s[2][pl.ds(i * 16, 16)] = s[0][0, pl.ds(i * 16, 16)] + roff
                    if ws_pred is None:
                        _wait_scat(s)
                    else:
                        @pl.when(ws_pred)
                        def _ws():
                            _wait_scat(s)
                    pltpu.async_copy(p_hbm.at[s[2]], s[4], s[6])

                def _scatter(b, s):
                    pltpu.make_async_copy(p_hbm.at[s[2]], s[4], s[6]).wait()
                    @pl.loop(0, _B // 16)
                    def _d(i):
                        s[3][pl.ds(i * 16, 16)] = s[1][0, pl.ds(i * 16, 16)]
                    pltpu.async_copy(s[4], acc_sh.at[s[3]], s[7], add=True)
                    @pl.when(b + 2 < nb)
                    def _ni():
                        _issue_idx(b + 2, s)

                @pl.loop(0, 39)
                def _acc(t):
                    b0 = 2 * t
                    _gather(b0, sets[0], t > 0)
                    _gather(b0 + 1, sets[1], t > 0)
                    _scatter(b0, sets[0])
                    _scatter(b0 + 1, sets[1])

                @pl.when(nb > 78)
                def _tail():
                    _gather(78, sets[0], None)
                    _scatter(78, sets[0])

                _wait_scat(sets[0])
                _wait_scat(sets[1])
                plsc.subcore_barrier()

                pltpu.sync_copy(acc_sh.at[pl.ds(sid * _SLAB, _SLAB)],
                                out_hbm.at[c, pl.ds(sid * _SLAB, _SLAB)])


@jax.jit
def _sc_aggregate(p_flat, src, dst):
    src3 = src.reshape(_NB, 1, _B)
    dst3 = dst.reshape(_NB, 1, _B)
    return pl.kernel(
        _agg_body,
        out_type=jax.ShapeDtypeStruct((_NCHUNK, _NPAD, _CW), jnp.float32),
        mesh=_mesh,
        scratch_types=[
            pltpu.VMEM((1, _B), jnp.int32),
            pltpu.VMEM((1, _B), jnp.int32),
            pltpu.VMEM((1, _B), jnp.int32),
            pltpu.VMEM((1, _B), jnp.int32),
            pltpu.VMEM((_B,), jnp.int32),
            pltpu.VMEM((_B,), jnp.int32),
            pltpu.VMEM((_B,), jnp.int32),
            pltpu.VMEM((_B,), jnp.int32),
            pltpu.VMEM((_B, _CW), jnp.float32),
            pltpu.VMEM((_B, _CW), jnp.float32),
            pltpu.VMEM_SHARED((_NPAD, _CW), jnp.float32),
            pltpu.SemaphoreType.DMA,
            pltpu.SemaphoreType.DMA,
            pltpu.SemaphoreType.DMA,
            pltpu.SemaphoreType.DMA,
            pltpu.SemaphoreType.DMA,
            pltpu.SemaphoreType.DMA,
        ],
    )(p_flat, src3, dst3)


_MB = 2000


def _mm_body(h_ref, wl_ref, wr_ref, b_ref, p_ref, q_ref):
    h = h_ref[...]
    for c in range(_NCHUNK):
        p_ref[c] = lax.dot_general(h, wl_ref[c], (((1,), (0,)), ((), ())),
                                   precision=lax.Precision.DEFAULT)
    q_ref[...] = lax.dot_general(h, wr_ref[...], (((1,), (0,)), ((), ())),
                                 precision=lax.Precision.DEFAULT) + b_ref[...]


def _tc_mm(h, wl_chunks, wr, b_row):
    k = h.shape[1]
    return pl.pallas_call(
        _mm_body,
        grid=(_N // _MB,),
        in_specs=[
            pl.BlockSpec((_MB, k), lambda i: (i, 0)),
            pl.BlockSpec((_NCHUNK, k, _CW), lambda i: (0, 0, 0)),
            pl.BlockSpec((k, _F), lambda i: (0, 0)),
            pl.BlockSpec((1, _F), lambda i: (0, 0)),
        ],
        out_specs=[
            pl.BlockSpec((_NCHUNK, _MB, _CW), lambda i: (0, i, 0)),
            pl.BlockSpec((_MB, _F), lambda i: (i, 0)),
        ],
        out_shape=[
            jax.ShapeDtypeStruct((_NCHUNK, _N, _CW), jnp.float32),
            jax.ShapeDtypeStruct((_N, _F), jnp.float32),
        ],
    )(h, wl_chunks, wr, b_row)


_MBC = 1000


def _combine_h(agg_ref, deg_ref, q_ref):
    deg = deg_ref[0, :, 0] + deg_ref[1, :, 0]
    rdeg = 1.0 / jnp.maximum(deg, 1.0)
    hs = []
    for c in range(_NCHUNK):
        v = agg_ref[c] * rdeg[:, None] + q_ref[:, c * _CW:(c + 1) * _CW]
        hs.append(jnp.maximum(v, 0.0))
    return jnp.concatenate(hs, axis=1)


def _cmm_body(agg_ref, deg_ref, q_ref, wl_ref, wr_ref, b_ref, p_ref, q_out_ref):
    h = _combine_h(agg_ref, deg_ref, q_ref)
    for c in range(_NCHUNK):
        p_ref[c] = lax.dot_general(h, wl_ref[c], (((1,), (0,)), ((), ())),
                                   precision=lax.Precision.DEFAULT)
    q_out_ref[...] = lax.dot_general(h, wr_ref[...], (((1,), (0,)), ((), ())),
                                     precision=lax.Precision.DEFAULT) + b_ref[...]


def _tc_cmm(agg, deg2, q, wl_chunks, wr, b_row):
    return pl.pallas_call(
        _cmm_body,
        grid=(_N // _MBC,),
        in_specs=[
            pl.BlockSpec((_NCHUNK, _MBC, _CW), lambda i: (0, i, 0)),
            pl.BlockSpec((_NCORE, _MBC, _CW), lambda i: (0, i, 0)),
            pl.BlockSpec((_MBC, _F), lambda i: (i, 0)),
            pl.BlockSpec((_NCHUNK, _F, _CW), lambda i: (0, 0, 0)),
            pl.BlockSpec((_F, _F), lambda i: (0, 0)),
            pl.BlockSpec((1, _F), lambda i: (0, 0)),
        ],
        out_specs=[
            pl.BlockSpec((_NCHUNK, _MBC, _CW), lambda i: (0, i, 0)),
            pl.BlockSpec((_MBC, _F), lambda i: (i, 0)),
        ],
        out_shape=[
            jax.ShapeDtypeStruct((_NCHUNK, _N, _CW), jnp.float32),
            jax.ShapeDtypeStruct((_N, _F), jnp.float32),
        ],
    )(agg, deg2, q, wl_chunks, wr, b_row)


def _cfin_body(agg_ref, deg_ref, q_ref, w_ref, b_ref, o_ref):
    h = _combine_h(agg_ref, deg_ref, q_ref)
    o_ref[...] = lax.dot_general(h, w_ref[...], (((1,), (0,)), ((), ())),
                                 precision=lax.Precision.DEFAULT) + b_ref[...]


def _tc_cfin(agg, deg2, q, w, b_row):
    nclass = w.shape[1]
    return pl.pallas_call(
        _cfin_body,
        grid=(_N // _MBC,),
        in_specs=[
            pl.BlockSpec((_NCHUNK, _MBC, _CW), lambda i: (0, i, 0)),
            pl.BlockSpec((_NCORE, _MBC, _CW), lambda i: (0, i, 0)),
            pl.BlockSpec((_MBC, _F), lambda i: (i, 0)),
            pl.BlockSpec((_F, nclass), lambda i: (0, 0)),
            pl.BlockSpec((1, nclass), lambda i: (0, 0)),
        ],
        out_specs=pl.BlockSpec((_MBC, nclass), lambda i: (i, 0)),
        out_shape=jax.ShapeDtypeStruct((_N, nclass), jnp.float32),
    )(agg, deg2, q, w, b_row)


def kernel(x, edge_index, edge_attr,
           W_l1, W_r1, b1, W_l2, W_r2, b2, W_l3, W_r3, b3,
           W_l4, W_r4, b4, W_l5, W_r5, b5, W_lin, b_lin):
    src = edge_index[0].astype(jnp.int32)
    dst = edge_index[1].astype(jnp.int32)

    deg2 = _sc_degree(dst)

    def wlc(Wl):
        k = Wl.shape[0]
        return jnp.transpose(Wl.reshape(k, _NCHUNK, _CW), (1, 0, 2))

    p4, q = _tc_mm(x, wlc(W_l1), W_r1, b1.reshape(1, _F))
    agg = _sc_aggregate(p4.reshape(_NCHUNK * _N, _CW), src, dst)
    for Wl, Wr, b in ((W_l2, W_r2, b2), (W_l3, W_r3, b3),
                      (W_l4, W_r4, b4), (W_l5, W_r5, b5)):
        p4, q = _tc_cmm(agg, deg2, q, wlc(Wl), Wr, b.reshape(1, _F))
        agg = _sc_aggregate(p4.reshape(_NCHUNK * _N, _CW), src, dst)

    return _tc_cfin(agg, deg2, q, W_lin, b_lin.reshape(1, W_lin.shape[1]))


# speedup vs baseline: 5.4618x; 1.0965x over previous
import functools

import jax
import jax.numpy as jnp
from jax import lax
from jax.experimental import pallas as pl
from jax.experimental.pallas import tpu as pltpu
from jax.experimental.pallas import tpu_sc as plsc

_N = 10000
_E = 160000
_F = 512
_CW = 128
_NCHUNK = _F // _CW
_B = 128
_NB = _E // _B
_NSUB = 16
_NCORE = 2
_NPAD = 10240
_SLAB = _NPAD // _NSUB

_mesh = plsc.VectorSubcoreMesh(core_axis_name="c", subcore_axis_name="s")


def _deg_body(dst_hbm, out_hbm,
              dstb0_v, dstb1_v, dstv0_v, dstv1_v, ones_v, acc_sh,
              isem0, isem1, ssem0, ssem1):
    cid = lax.axis_index("c")
    sid = lax.axis_index("s")
    nb = jnp.where(sid < 1, 40, 39)
    bstart = sid * 39 + jnp.minimum(sid, 1)

    sets = ((dstb0_v, dstv0_v, isem0, ssem0),
            (dstb1_v, dstv1_v, isem1, ssem1))

    for core_k in range(_NCORE):
        @pl.when(cid == core_k)
        def _core():
            gb0 = core_k * (_NB // 2) + bstart

            def _issue_idx(b, s):
                pltpu.async_copy(dst_hbm.at[gb0 + b], s[0], s[2])

            def _wait_scat(s):
                pltpu.make_async_copy(ones_v, acc_sh.at[s[1]], s[3]).wait()

            _issue_idx(0, sets[0])
            _issue_idx(1, sets[1])

            @pl.loop(0, _B)
            def _mk0(r):
                @pl.loop(0, _CW // 16)
                def _mk0i(i):
                    ones_v[r, pl.ds(i * 16, 16)] = jnp.zeros((16,), jnp.float32)

            @pl.loop(0, _SLAB // _B)
            def _z(t):
                pltpu.sync_copy(ones_v, acc_sh.at[pl.ds(sid * _SLAB + t * _B, _B)])
            plsc.subcore_barrier()

            @pl.loop(0, _B)
            def _mk1(r):
                @pl.loop(0, _CW // 16)
                def _mk1i(i):
                    ones_v[r, pl.ds(i * 16, 16)] = jnp.ones((16,), jnp.float32)

            def _step(b, s, ws_pred):
                pltpu.make_async_copy(dst_hbm.at[0], s[0], s[2]).wait()
                @pl.loop(0, _B // 16)
                def _d(i):
                    s[1][pl.ds(i * 16, 16)] = s[0][0, pl.ds(i * 16, 16)]
                if ws_pred is not None:
                    @pl.when(ws_pred)
                    def _ws():
                        _wait_scat(s)
                pltpu.async_copy(ones_v, acc_sh.at[s[1]], s[3], add=True)
                @pl.when(b + 2 < nb)
                def _ni():
                    _issue_idx(b + 2, s)

            @pl.loop(0, 19)
            def _acc(t):
                b0 = 2 * t
                _step(b0, sets[0], t > 0)
                _step(b0 + 1, sets[1], t > 0)

            _step(38, sets[0], jnp.bool_(True))
            @pl.when(nb > 39)
            def _tail():
                _step(39, sets[1], jnp.bool_(True))

            _wait_scat(sets[0])
            _wait_scat(sets[1])
            plsc.subcore_barrier()

            pltpu.sync_copy(acc_sh.at[pl.ds(sid * _SLAB, _SLAB)],
                            out_hbm.at[core_k, pl.ds(sid * _SLAB, _SLAB)])


@jax.jit
def _sc_degree(dst):
    dst3 = dst.reshape(_NB, 1, _B)
    return pl.kernel(
        _deg_body,
        out_type=jax.ShapeDtypeStruct((_NCORE, _NPAD, _CW), jnp.float32),
        mesh=_mesh,
        scratch_types=[
            pltpu.VMEM((1, _B), jnp.int32),
            pltpu.VMEM((1, _B), jnp.int32),
            pltpu.VMEM((_B,), jnp.int32),
            pltpu.VMEM((_B,), jnp.int32),
            pltpu.VMEM((_B, _CW), jnp.float32),
            pltpu.VMEM_SHARED((_NPAD, _CW), jnp.float32),
            pltpu.SemaphoreType.DMA,
            pltpu.SemaphoreType.DMA,
            pltpu.SemaphoreType.DMA,
            pltpu.SemaphoreType.DMA,
        ],
    )(dst3)


def _agg_body(cpc, p_hbm, src_hbm, dst_hbm, out_hbm,
              srcb0_v, dstb0_v, srcb1_v, dstb1_v,
              adj0_v, adj1_v, dstv0_v, dstv1_v,
              rows0_v, rows1_v, acc_sh,
              isem0, isem1, gsem0, gsem1, ssem0, ssem1):
    cid = lax.axis_index("c")
    sid = lax.axis_index("s")
    nb = jnp.where(sid < 2, 79, 78)
    bstart = sid * 78 + jnp.minimum(sid, 2)

    sets = ((srcb0_v, dstb0_v, adj0_v, dstv0_v, rows0_v, isem0, gsem0, ssem0),
            (srcb1_v, dstb1_v, adj1_v, dstv1_v, rows1_v, isem1, gsem1, ssem1))

    def _issue_idx(b, s):
        srcb, dstb = s[0], s[1]
        pltpu.async_copy(src_hbm.at[bstart + b], srcb, s[5])
        pltpu.async_copy(dst_hbm.at[bstart + b], dstb, s[5])

    def _wait_idx(s):
        pltpu.make_async_copy(src_hbm.at[0], s[0], s[5]).wait()
        pltpu.make_async_copy(dst_hbm.at[0], s[1], s[5]).wait()

    def _wait_scat(s):
        pltpu.make_async_copy(s[4], acc_sh.at[s[3]], s[7]).wait()

    for core_k in range(_NCORE):
        @pl.when(cid == core_k)
        def _core():
            for j in range(cpc):
                c = core_k * cpc + j
                roff = c * _N

                _issue_idx(0, sets[0])
                _issue_idx(1, sets[1])

                @pl.loop(0, _B)
                def _mk0(r):
                    @pl.loop(0, _CW // 16)
                    def _mk0i(i):
                        rows0_v[r, pl.ds(i * 16, 16)] = jnp.zeros((16,), jnp.float32)

                @pl.loop(0, _SLAB // _B)
                def _z(t):
                    pltpu.sync_copy(rows0_v, acc_sh.at[pl.ds(sid * _SLAB + t * _B, _B)])
                plsc.subcore_barrier()

                def _gather(b, s, ws_pred):
                    _wait_idx(s)
                    @pl.loop(0, _B // 16)
                    def _adj(i):
                        s[2][pl.ds(i * 16, 16)] = s[0][0, pl.ds(i * 16, 16)] + roff
                    if ws_pred is None:
                        _wait_scat(s)
                    else:
                        @pl.when(ws_pred)
                        def _ws():
                            _wait_scat(s)
                    pltpu.async_copy(p_hbm.at[s[2]], s[4], s[6])

                def _scatter(b, s):
                    pltpu.make_async_copy(p_hbm.at[s[2]], s[4], s[6]).wait()
                    @pl.loop(0, _B // 16)
                    def _d(i):
                        s[3][pl.ds(i * 16, 16)] = s[1][0, pl.ds(i * 16, 16)]
                    pltpu.async_copy(s[4], acc_sh.at[s[3]], s[7], add=True)
                    @pl.when(b + 2 < nb)
                    def _ni():
                        _issue_idx(b + 2, s)

                @pl.loop(0, 39)
                def _acc(t):
                    b0 = 2 * t
                    _gather(b0, sets[0], t > 0)
                    _gather(b0 + 1, sets[1], t > 0)
                    _scatter(b0, sets[0])
                    _scatter(b0 + 1, sets[1])

                @pl.when(nb > 78)
                def _tail():
                    _gather(78, sets[0], None)
                    _scatter(78, sets[0])

                _wait_scat(sets[0])
                _wait_scat(sets[1])
                plsc.subcore_barrier()

                pltpu.sync_copy(acc_sh.at[pl.ds(sid * _SLAB, _SLAB)],
                                out_hbm.at[c, pl.ds(sid * _SLAB, _SLAB)])


@functools.partial(jax.jit, static_argnums=(3,))
def _sc_aggregate(p_flat, src, dst, nchunk=_NCHUNK):
    src3 = src.reshape(_NB, 1, _B)
    dst3 = dst.reshape(_NB, 1, _B)
    return pl.kernel(
        functools.partial(_agg_body, nchunk // _NCORE),
        out_type=jax.ShapeDtypeStruct((nchunk, _NPAD, _CW), jnp.float32),
        mesh=_mesh,
        scratch_types=[
            pltpu.VMEM((1, _B), jnp.int32),
            pltpu.VMEM((1, _B), jnp.int32),
            pltpu.VMEM((1, _B), jnp.int32),
            pltpu.VMEM((1, _B), jnp.int32),
            pltpu.VMEM((_B,), jnp.int32),
            pltpu.VMEM((_B,), jnp.int32),
            pltpu.VMEM((_B,), jnp.int32),
            pltpu.VMEM((_B,), jnp.int32),
            pltpu.VMEM((_B, _CW), jnp.float32),
            pltpu.VMEM((_B, _CW), jnp.float32),
            pltpu.VMEM_SHARED((_NPAD, _CW), jnp.float32),
            pltpu.SemaphoreType.DMA,
            pltpu.SemaphoreType.DMA,
            pltpu.SemaphoreType.DMA,
            pltpu.SemaphoreType.DMA,
            pltpu.SemaphoreType.DMA,
            pltpu.SemaphoreType.DMA,
        ],
    )(p_flat, src3, dst3)


_MB = 2000


def _q_body(x_ref, wr_ref, b_ref, q_ref):
    q_ref[...] = lax.dot_general(x_ref[...], wr_ref[...],
                                 (((1,), (0,)), ((), ())),
                                 precision=lax.Precision.DEFAULT) + b_ref[...]


def _tc_q(x, wr, b_row):
    k = x.shape[1]
    return pl.pallas_call(
        _q_body,
        grid=(_N // _MB,),
        in_specs=[
            pl.BlockSpec((_MB, k), lambda i: (i, 0)),
            pl.BlockSpec((k, _F), lambda i: (0, 0)),
            pl.BlockSpec((1, _F), lambda i: (0, 0)),
        ],
        out_specs=pl.BlockSpec((_MB, _F), lambda i: (i, 0)),
        out_shape=jax.ShapeDtypeStruct((_N, _F), jnp.float32),
    )(x, wr, b_row)


_MBC = 1000


def _combine_h(agg_ref, deg_ref, q_ref):
    deg = deg_ref[0, :, 0] + deg_ref[1, :, 0]
    rdeg = 1.0 / jnp.maximum(deg, 1.0)
    hs = []
    for c in range(_NCHUNK):
        v = agg_ref[c] * rdeg[:, None] + q_ref[:, c * _CW:(c + 1) * _CW]
        hs.append(jnp.maximum(v, 0.0))
    return jnp.concatenate(hs, axis=1)


def _cmm_body(agg_ref, deg_ref, q_ref, wl_ref, wr_ref, b_ref, p_ref, q_out_ref):
    h = _combine_h(agg_ref, deg_ref, q_ref)
    for c in range(_NCHUNK):
        p_ref[c] = lax.dot_general(h, wl_ref[c], (((1,), (0,)), ((), ())),
                                   precision=lax.Precision.DEFAULT)
    q_out_ref[...] = lax.dot_general(h, wr_ref[...], (((1,), (0,)), ((), ())),
                                     precision=lax.Precision.DEFAULT) + b_ref[...]


def _tc_cmm(agg, deg2, q, wl_chunks, wr, b_row):
    return pl.pallas_call(
        _cmm_body,
        grid=(_N // _MBC,),
        in_specs=[
            pl.BlockSpec((_NCHUNK, _MBC, _CW), lambda i: (0, i, 0)),
            pl.BlockSpec((_NCORE, _MBC, _CW), lambda i: (0, i, 0)),
            pl.BlockSpec((_MBC, _F), lambda i: (i, 0)),
            pl.BlockSpec((_NCHUNK, _F, _CW), lambda i: (0, 0, 0)),
            pl.BlockSpec((_F, _F), lambda i: (0, 0)),
            pl.BlockSpec((1, _F), lambda i: (0, 0)),
        ],
        out_specs=[
            pl.BlockSpec((_NCHUNK, _MBC, _CW), lambda i: (0, i, 0)),
            pl.BlockSpec((_MBC, _F), lambda i: (i, 0)),
        ],
        out_shape=[
            jax.ShapeDtypeStruct((_NCHUNK, _N, _CW), jnp.float32),
            jax.ShapeDtypeStruct((_N, _F), jnp.float32),
        ],
    )(agg, deg2, q, wl_chunks, wr, b_row)


def _c1mm_body(aggx_ref, deg_ref, q_ref, wl1_ref, wl_ref, wr_ref, b_ref,
               p_ref, q_out_ref):
    deg = deg_ref[0, :, 0] + deg_ref[1, :, 0]
    rdeg = 1.0 / jnp.maximum(deg, 1.0)
    m = jnp.concatenate([aggx_ref[0], aggx_ref[1]], axis=1) * rdeg[:, None]
    h = lax.dot_general(m, wl1_ref[...], (((1,), (0,)), ((), ())),
                        precision=lax.Precision.DEFAULT) + q_ref[...]
    h = jnp.maximum(h, 0.0)
    for c in range(_NCHUNK):
        p_ref[c] = lax.dot_general(h, wl_ref[c], (((1,), (0,)), ((), ())),
                                   precision=lax.Precision.DEFAULT)
    q_out_ref[...] = lax.dot_general(h, wr_ref[...], (((1,), (0,)), ((), ())),
                                     precision=lax.Precision.DEFAULT) + b_ref[...]


def _tc_c1mm(aggx, deg2, q1, wl1, wl_chunks, wr, b_row):
    kin = wl1.shape[0]
    return pl.pallas_call(
        _c1mm_body,
        grid=(_N // _MBC,),
        in_specs=[
            pl.BlockSpec((2, _MBC, _CW), lambda i: (0, i, 0)),
            pl.BlockSpec((_NCORE, _MBC, _CW), lambda i: (0, i, 0)),
            pl.BlockSpec((_MBC, _F), lambda i: (i, 0)),
            pl.BlockSpec((kin, _F), lambda i: (0, 0)),
            pl.BlockSpec((_NCHUNK, _F, _CW), lambda i: (0, 0, 0)),
            pl.BlockSpec((_F, _F), lambda i: (0, 0)),
            pl.BlockSpec((1, _F), lambda i: (0, 0)),
        ],
        out_specs=[
            pl.BlockSpec((_NCHUNK, _MBC, _CW), lambda i: (0, i, 0)),
            pl.BlockSpec((_MBC, _F), lambda i: (i, 0)),
        ],
        out_shape=[
            jax.ShapeDtypeStruct((_NCHUNK, _N, _CW), jnp.float32),
            jax.ShapeDtypeStruct((_N, _F), jnp.float32),
        ],
    )(aggx, deg2, q1, wl1, wl_chunks, wr, b_row)


def _cfin_body(agg_ref, deg_ref, q_ref, w_ref, b_ref, o_ref):
    h = _combine_h(agg_ref, deg_ref, q_ref)
    o_ref[...] = lax.dot_general(h, w_ref[...], (((1,), (0,)), ((), ())),
                                 precision=lax.Precision.DEFAULT) + b_ref[...]


def _tc_cfin(agg, deg2, q, w, b_row):
    nclass = w.shape[1]
    return pl.pallas_call(
        _cfin_body,
        grid=(_N // _MBC,),
        in_specs=[
            pl.BlockSpec((_NCHUNK, _MBC, _CW), lambda i: (0, i, 0)),
            pl.BlockSpec((_NCORE, _MBC, _CW), lambda i: (0, i, 0)),
            pl.BlockSpec((_MBC, _F), lambda i: (i, 0)),
            pl.BlockSpec((_F, nclass), lambda i: (0, 0)),
            pl.BlockSpec((1, nclass), lambda i: (0, 0)),
        ],
        out_specs=pl.BlockSpec((_MBC, nclass), lambda i: (i, 0)),
        out_shape=jax.ShapeDtypeStruct((_N, nclass), jnp.float32),
    )(agg, deg2, q, w, b_row)


def kernel(x, edge_index, edge_attr,
           W_l1, W_r1, b1, W_l2, W_r2, b2, W_l3, W_r3, b3,
           W_l4, W_r4, b4, W_l5, W_r5, b5, W_lin, b_lin):
    src = edge_index[0].astype(jnp.int32)
    dst = edge_index[1].astype(jnp.int32)

    deg2 = _sc_degree(dst)

    def wlc(Wl):
        k = Wl.shape[0]
        return jnp.transpose(Wl.reshape(k, _NCHUNK, _CW), (1, 0, 2))

    x2 = jnp.transpose(x.reshape(_N, 2, _CW), (1, 0, 2)).reshape(2 * _N, _CW)
    aggx = _sc_aggregate(x2, src, dst, 2)
    q = _tc_q(x, W_r1, b1.reshape(1, _F))
    p4, q = _tc_c1mm(aggx, deg2, q, W_l1, wlc(W_l2), W_r2, b2.reshape(1, _F))
    agg = _sc_aggregate(p4.reshape(_NCHUNK * _N, _CW), src, dst)
    for Wl, Wr, b in ((W_l3, W_r3, b3), (W_l4, W_r4, b4), (W_l5, W_r5, b5)):
        p4, q = _tc_cmm(agg, deg2, q, wlc(Wl), Wr, b.reshape(1, _F))
        agg = _sc_aggregate(p4.reshape(_NCHUNK * _N, _CW), src, dst)

    return _tc_cfin(agg, deg2, q, W_lin, b_lin.reshape(1, W_lin.shape[1]))


# speedup vs baseline: 5.4871x; 1.0046x over previous
import functools

import jax
import jax.numpy as jnp
from jax import lax
from jax.experimental import pallas as pl
from jax.experimental.pallas import tpu as pltpu
from jax.experimental.pallas import tpu_sc as plsc

_N = 10000
_E = 160000
_F = 512
_CW = 128
_NCHUNK = _F // _CW
_B = 128
_NB = _E // _B
_NSUB = 16
_NCORE = 2
_NPAD = 10240
_SLAB = _NPAD // _NSUB

_mesh = plsc.VectorSubcoreMesh(core_axis_name="c", subcore_axis_name="s")


def _deg_body(dst_hbm, out_hbm,
              dstb0_v, dstb1_v, dstv0_v, dstv1_v, ones_v, acc_sh,
              isem0, isem1, ssem0, ssem1):
    cid = lax.axis_index("c")
    sid = lax.axis_index("s")
    nb = jnp.where(sid < 1, 40, 39)
    bstart = sid * 39 + jnp.minimum(sid, 1)

    sets = ((dstb0_v, dstv0_v, isem0, ssem0),
            (dstb1_v, dstv1_v, isem1, ssem1))

    for core_k in range(_NCORE):
        @pl.when(cid == core_k)
        def _core():
            gb0 = core_k * (_NB // 2) + bstart

            def _issue_idx(b, s):
                pltpu.async_copy(dst_hbm.at[gb0 + b], s[0], s[2])

            def _wait_scat(s):
                pltpu.make_async_copy(ones_v, acc_sh.at[s[1]], s[3]).wait()

            _issue_idx(0, sets[0])
            _issue_idx(1, sets[1])

            @pl.loop(0, _B)
            def _mk0(r):
                @pl.loop(0, _CW // 16)
                def _mk0i(i):
                    ones_v[r, pl.ds(i * 16, 16)] = jnp.zeros((16,), jnp.float32)

            @pl.loop(0, _SLAB // _B)
            def _z(t):
                pltpu.sync_copy(ones_v, acc_sh.at[pl.ds(sid * _SLAB + t * _B, _B)])
            plsc.subcore_barrier()

            @pl.loop(0, _B)
            def _mk1(r):
                @pl.loop(0, _CW // 16)
                def _mk1i(i):
                    ones_v[r, pl.ds(i * 16, 16)] = jnp.ones((16,), jnp.float32)

            def _step(b, s, ws_pred):
                pltpu.make_async_copy(dst_hbm.at[0], s[0], s[2]).wait()
                @pl.loop(0, _B // 16)
                def _d(i):
                    s[1][pl.ds(i * 16, 16)] = s[0][0, pl.ds(i * 16, 16)]
                if ws_pred is not None:
                    @pl.when(ws_pred)
                    def _ws():
                        _wait_scat(s)
                pltpu.async_copy(ones_v, acc_sh.at[s[1]], s[3], add=True)
                @pl.when(b + 2 < nb)
                def _ni():
                    _issue_idx(b + 2, s)

            @pl.loop(0, 19)
            def _acc(t):
                b0 = 2 * t
                _step(b0, sets[0], t > 0)
                _step(b0 + 1, sets[1], t > 0)

            _step(38, sets[0], jnp.bool_(True))
            @pl.when(nb > 39)
            def _tail():
                _step(39, sets[1], jnp.bool_(True))

            _wait_scat(sets[0])
            _wait_scat(sets[1])
            plsc.subcore_barrier()

            pltpu.sync_copy(acc_sh.at[pl.ds(sid * _SLAB, _SLAB)],
                            out_hbm.at[core_k, pl.ds(sid * _SLAB, _SLAB)])


@jax.jit
def _sc_degree(dst):
    dst3 = dst.reshape(_NB, 1, _B)
    return pl.kernel(
        _deg_body,
        out_type=jax.ShapeDtypeStruct((_NCORE, _NPAD, _CW), jnp.float32),
        mesh=_mesh,
        scratch_types=[
            pltpu.VMEM((1, _B), jnp.int32),
            pltpu.VMEM((1, _B), jnp.int32),
            pltpu.VMEM((_B,), jnp.int32),
            pltpu.VMEM((_B,), jnp.int32),
            pltpu.VMEM((_B, _CW), jnp.float32),
            pltpu.VMEM_SHARED((_NPAD, _CW), jnp.float32),
            pltpu.SemaphoreType.DMA,
            pltpu.SemaphoreType.DMA,
            pltpu.SemaphoreType.DMA,
            pltpu.SemaphoreType.DMA,
        ],
    )(dst3)


def _agg_body(cpc, p_hbm, edg_hbm, out_hbm,
              eb0_v, eb1_v,
              adj0_v, adj1_v, dstv0_v, dstv1_v,
              rows0_v, rows1_v, acc_sh,
              isem0, isem1, gsem0, gsem1, ssem0, ssem1):
    cid = lax.axis_index("c")
    sid = lax.axis_index("s")
    nb = jnp.where(sid < 2, 79, 78)
    bstart = sid * 78 + jnp.minimum(sid, 2)

    sets = ((eb0_v, adj0_v, dstv0_v, rows0_v, isem0, gsem0, ssem0),
            (eb1_v, adj1_v, dstv1_v, rows1_v, isem1, gsem1, ssem1))

    def _issue_idx(b, s):
        pltpu.async_copy(edg_hbm.at[bstart + b], s[0], s[4])

    def _wait_idx(s):
        pltpu.make_async_copy(edg_hbm.at[0], s[0], s[4]).wait()

    def _wait_scat(s):
        pltpu.make_async_copy(s[3], acc_sh.at[s[2]], s[6]).wait()

    for core_k in range(_NCORE):
        @pl.when(cid == core_k)
        def _core():
            for j in range(cpc):
                c = core_k * cpc + j
                roff = c * _N

                _issue_idx(0, sets[0])
                _issue_idx(1, sets[1])

                @pl.loop(0, _B)
                def _mk0(r):
                    @pl.loop(0, _CW // 16)
                    def _mk0i(i):
                        rows0_v[r, pl.ds(i * 16, 16)] = jnp.zeros((16,), jnp.float32)

                @pl.loop(0, _SLAB // _B)
                def _z(t):
                    pltpu.sync_copy(rows0_v, acc_sh.at[pl.ds(sid * _SLAB + t * _B, _B)])
                plsc.subcore_barrier()

                def _gather(b, s, ws_pred):
                    _wait_idx(s)
                    @pl.loop(0, _B // 16)
                    def _adj(i):
                        s[1][pl.ds(i * 16, 16)] = s[0][0, pl.ds(i * 16, 16)] + roff
                    if ws_pred is None:
                        _wait_scat(s)
                    else:
                        @pl.when(ws_pred)
                        def _ws():
                            _wait_scat(s)
                    pltpu.async_copy(p_hbm.at[s[1]], s[3], s[5])

                def _scatter(b, s):
                    pltpu.make_async_copy(p_hbm.at[s[1]], s[3], s[5]).wait()
                    @pl.loop(0, _B // 16)
                    def _d(i):
                        s[2][pl.ds(i * 16, 16)] = s[0][1, pl.ds(i * 16, 16)]
                    pltpu.async_copy(s[3], acc_sh.at[s[2]], s[6], add=True)
                    @pl.when(b + 2 < nb)
                    def _ni():
                        _issue_idx(b + 2, s)

                @pl.loop(0, 39)
                def _acc(t):
                    b0 = 2 * t
                    _gather(b0, sets[0], t > 0)
                    _gather(b0 + 1, sets[1], t > 0)
                    _scatter(b0, sets[0])
                    _scatter(b0 + 1, sets[1])

                @pl.when(nb > 78)
                def _tail():
                    _gather(78, sets[0], None)
                    _scatter(78, sets[0])

                _wait_scat(sets[0])
                _wait_scat(sets[1])
                plsc.subcore_barrier()

                pltpu.sync_copy(acc_sh.at[pl.ds(sid * _SLAB, _SLAB)],
                                out_hbm.at[c, pl.ds(sid * _SLAB, _SLAB)])


@functools.partial(jax.jit, static_argnums=(2,))
def _sc_aggregate(p_flat, edg, nchunk=_NCHUNK):
    return pl.kernel(
        functools.partial(_agg_body, nchunk // _NCORE),
        out_type=jax.ShapeDtypeStruct((nchunk, _NPAD, _CW), jnp.float32),
        mesh=_mesh,
        scratch_types=[
            pltpu.VMEM((2, _B), jnp.int32),
            pltpu.VMEM((2, _B), jnp.int32),
            pltpu.VMEM((_B,), jnp.int32),
            pltpu.VMEM((_B,), jnp.int32),
            pltpu.VMEM((_B,), jnp.int32),
            pltpu.VMEM((_B,), jnp.int32),
            pltpu.VMEM((_B, _CW), jnp.float32),
            pltpu.VMEM((_B, _CW), jnp.float32),
            pltpu.VMEM_SHARED((_NPAD, _CW), jnp.float32),
            pltpu.SemaphoreType.DMA,
            pltpu.SemaphoreType.DMA,
            pltpu.SemaphoreType.DMA,
            pltpu.SemaphoreType.DMA,
            pltpu.SemaphoreType.DMA,
            pltpu.SemaphoreType.DMA,
        ],
    )(p_flat, edg)


_MB = 2000


def _q_body(x_ref, wr_ref, b_ref, q_ref):
    q_ref[...] = lax.dot_general(x_ref[...], wr_ref[...],
                                 (((1,), (0,)), ((), ())),
                                 precision=lax.Precision.DEFAULT) + b_ref[...]


def _tc_q(x, wr, b_row):
    k = x.shape[1]
    return pl.pallas_call(
        _q_body,
        grid=(_N // _MB,),
        in_specs=[
            pl.BlockSpec((_MB, k), lambda i: (i, 0)),
            pl.BlockSpec((k, _F), lambda i: (0, 0)),
            pl.BlockSpec((1, _F), lambda i: (0, 0)),
        ],
        out_specs=pl.BlockSpec((_MB, _F), lambda i: (i, 0)),
        out_shape=jax.ShapeDtypeStruct((_N, _F), jnp.float32),
    )(x, wr, b_row)


_MBC = 1000


def _combine_h(agg_ref, deg_ref, q_ref):
    deg = deg_ref[0, :, 0] + deg_ref[1, :, 0]
    rdeg = 1.0 / jnp.maximum(deg, 1.0)
    hs = []
    for c in range(_NCHUNK):
        v = agg_ref[c] * rdeg[:, None] + q_ref[:, c * _CW:(c + 1) * _CW]
        hs.append(jnp.maximum(v, 0.0))
    return jnp.concatenate(hs, axis=1)


def _cmm_body(agg_ref, deg_ref, q_ref, wl_ref, wr_ref, b_ref, p_ref, q_out_ref):
    h = _combine_h(agg_ref, deg_ref, q_ref)
    for c in range(_NCHUNK):
        p_ref[c] = lax.dot_general(h, wl_ref[c], (((1,), (0,)), ((), ())),
                                   precision=lax.Precision.DEFAULT)
    q_out_ref[...] = lax.dot_general(h, wr_ref[...], (((1,), (0,)), ((), ())),
                                     precision=lax.Precision.DEFAULT) + b_ref[...]


def _tc_cmm(agg, deg2, q, wl_chunks, wr, b_row):
    return pl.pallas_call(
        _cmm_body,
        grid=(_N // _MBC,),
        in_specs=[
            pl.BlockSpec((_NCHUNK, _MBC, _CW), lambda i: (0, i, 0)),
            pl.BlockSpec((_NCORE, _MBC, _CW), lambda i: (0, i, 0)),
            pl.BlockSpec((_MBC, _F), lambda i: (i, 0)),
            pl.BlockSpec((_NCHUNK, _F, _CW), lambda i: (0, 0, 0)),
            pl.BlockSpec((_F, _F), lambda i: (0, 0)),
            pl.BlockSpec((1, _F), lambda i: (0, 0)),
        ],
        out_specs=[
            pl.BlockSpec((_NCHUNK, _MBC, _CW), lambda i: (0, i, 0)),
            pl.BlockSpec((_MBC, _F), lambda i: (i, 0)),
        ],
        out_shape=[
            jax.ShapeDtypeStruct((_NCHUNK, _N, _CW), jnp.float32),
            jax.ShapeDtypeStruct((_N, _F), jnp.float32),
        ],
    )(agg, deg2, q, wl_chunks, wr, b_row)


def _c1mm_body(aggx_ref, deg_ref, q_ref, wl1_ref, wl_ref, wr_ref, b_ref,
               p_ref, q_out_ref):
    deg = deg_ref[0, :, 0] + deg_ref[1, :, 0]
    rdeg = 1.0 / jnp.maximum(deg, 1.0)
    m = jnp.concatenate([aggx_ref[0], aggx_ref[1]], axis=1) * rdeg[:, None]
    h = lax.dot_general(m, wl1_ref[...], (((1,), (0,)), ((), ())),
                        precision=lax.Precision.DEFAULT) + q_ref[...]
    h = jnp.maximum(h, 0.0)
    for c in range(_NCHUNK):
        p_ref[c] = lax.dot_general(h, wl_ref[c], (((1,), (0,)), ((), ())),
                                   precision=lax.Precision.DEFAULT)
    q_out_ref[...] = lax.dot_general(h, wr_ref[...], (((1,), (0,)), ((), ())),
                                     precision=lax.Precision.DEFAULT) + b_ref[...]


def _tc_c1mm(aggx, deg2, q1, wl1, wl_chunks, wr, b_row):
    kin = wl1.shape[0]
    return pl.pallas_call(
        _c1mm_body,
        grid=(_N // _MBC,),
        in_specs=[
            pl.BlockSpec((2, _MBC, _CW), lambda i: (0, i, 0)),
            pl.BlockSpec((_NCORE, _MBC, _CW), lambda i: (0, i, 0)),
            pl.BlockSpec((_MBC, _F), lambda i: (i, 0)),
            pl.BlockSpec((kin, _F), lambda i: (0, 0)),
            pl.BlockSpec((_NCHUNK, _F, _CW), lambda i: (0, 0, 0)),
            pl.BlockSpec((_F, _F), lambda i: (0, 0)),
            pl.BlockSpec((1, _F), lambda i: (0, 0)),
        ],
        out_specs=[
            pl.BlockSpec((_NCHUNK, _MBC, _CW), lambda i: (0, i, 0)),
            pl.BlockSpec((_MBC, _F), lambda i: (i, 0)),
        ],
        out_shape=[
            jax.ShapeDtypeStruct((_NCHUNK, _N, _CW), jnp.float32),
            jax.ShapeDtypeStruct((_N, _F), jnp.float32),
        ],
    )(aggx, deg2, q1, wl1, wl_chunks, wr, b_row)


def _cfin_body(agg_ref, deg_ref, q_ref, w_ref, b_ref, o_ref):
    h = _combine_h(agg_ref, deg_ref, q_ref)
    o_ref[...] = lax.dot_general(h, w_ref[...], (((1,), (0,)), ((), ())),
                                 precision=lax.Precision.DEFAULT) + b_ref[...]


def _tc_cfin(agg, deg2, q, w, b_row):
    nclass = w.shape[1]
    return pl.pallas_call(
        _cfin_body,
        grid=(_N // _MBC,),
        in_specs=[
            pl.BlockSpec((_NCHUNK, _MBC, _CW), lambda i: (0, i, 0)),
            pl.BlockSpec((_NCORE, _MBC, _CW), lambda i: (0, i, 0)),
            pl.BlockSpec((_MBC, _F), lambda i: (i, 0)),
            pl.BlockSpec((_F, nclass), lambda i: (0, 0)),
            pl.BlockSpec((1, nclass), lambda i: (0, 0)),
        ],
        out_specs=pl.BlockSpec((_MBC, nclass), lambda i: (i, 0)),
        out_shape=jax.ShapeDtypeStruct((_N, nclass), jnp.float32),
    )(agg, deg2, q, w, b_row)


def kernel(x, edge_index, edge_attr,
           W_l1, W_r1, b1, W_l2, W_r2, b2, W_l3, W_r3, b3,
           W_l4, W_r4, b4, W_l5, W_r5, b5, W_lin, b_lin):
    dst = edge_index[1].astype(jnp.int32)
    edg = jnp.transpose(edge_index.astype(jnp.int32).reshape(2, _NB, _B),
                        (1, 0, 2))

    deg2 = _sc_degree(dst)

    def wlc(Wl):
        k = Wl.shape[0]
        return jnp.transpose(Wl.reshape(k, _NCHUNK, _CW), (1, 0, 2))

    x2 = jnp.transpose(x.reshape(_N, 2, _CW), (1, 0, 2)).reshape(2 * _N, _CW)
    aggx = _sc_aggregate(x2, edg, 2)
    q = _tc_q(x, W_r1, b1.reshape(1, _F))
    p4, q = _tc_c1mm(aggx, deg2, q, W_l1, wlc(W_l2), W_r2, b2.reshape(1, _F))
    agg = _sc_aggregate(p4.reshape(_NCHUNK * _N, _CW), edg)
    for Wl, Wr, b in ((W_l3, W_r3, b3), (W_l4, W_r4, b4), (W_l5, W_r5, b5)):
        p4, q = _tc_cmm(agg, deg2, q, wlc(Wl), Wr, b.reshape(1, _F))
        agg = _sc_aggregate(p4.reshape(_NCHUNK * _N, _CW), edg)

    return _tc_cfin(agg, deg2, q, W_lin, b_lin.reshape(1, W_lin.shape[1]))
